# Initial kernel scaffold; baseline (speedup 1.0000x reference)
import jax
import jax.numpy as jnp
from jax.experimental import pallas as pl


def kernel(x0, x1, x2, mW0_1, mb0_1, mW0_2, mb0_2, mW1_1, mb1_1, mW1_2, mb1_2, mW2_1, mb2_1, mW2_2, mb2_2, attn_weights, fW, fb):
    raise NotImplementedError("write your pallas kernel here")


# speedup vs baseline: 4.3984x; 4.3984x over previous
import functools
import jax
import jax.numpy as jnp
from jax import lax
from jax.experimental import pallas as pl

N = 4096
HID = 256
TOP_K = 10
BLK_A = 512
BLK_B = 256
BLK_C = 256
SENT = -1e9


def _dense_body(x0, x1, x2, w01, w02, w11, w12, w21, w22, aw, fw, fb,
                fused_out, normed_out, rsum_out):
    a = aw[...]
    a = a - jnp.max(a, axis=1, keepdims=True)
    e = jnp.exp(a)
    a = e / jnp.sum(e, axis=1, keepdims=True)

    def mlp(x, w1, w2):
        h = jnp.maximum(jnp.dot(x[...], w1[...], preferred_element_type=jnp.float32), 0.0)
        return jnp.dot(h, w2[...], preferred_element_type=jnp.float32)

    f0 = mlp(x0, w01, w02)
    f1 = mlp(x1, w11, w12)
    f2 = mlp(x2, w21, w22)
    fsum = a[0, 0] * f0 + a[0, 1] * f1 + a[0, 2] * f2
    fused = jnp.dot(fsum, fw[...], preferred_element_type=jnp.float32) + fb[...]
    fused_out[...] = fused
    nrm = jnp.sqrt(jnp.sum(fused * fused, axis=1, keepdims=True))
    nrm = jnp.maximum(nrm, 1e-12)
    normed_out[...] = fused / nrm
    rsum_out[...] = jnp.sum(fused, axis=1, keepdims=True)


def _topk_body(nb, nt, rsum, idx_out, nbr_out, vals_out, w_out):
    i = pl.program_id(0)
    r0 = i * BLK_B
    rids = (r0 + lax.broadcasted_iota(jnp.int32, (BLK_B, 1), 0)).astype(jnp.float32)
    cols = lax.broadcasted_iota(jnp.int32, (BLK_B, N), 1).astype(jnp.float32)
    sim = jnp.dot(nb[...], nt[...], preferred_element_type=jnp.float32)
    sim = jnp.where(cols == rids, SENT, sim)
    picks = []
    for _ in range(TOP_K):
        m = jnp.max(sim, axis=1, keepdims=True)
        cand = jnp.where(sim == m, cols, float(N))
        j = jnp.min(cand, axis=1, keepdims=True)
        sim = jnp.where(cols == j, SENT, sim)
        picks.append(j)
    selmask = (sim == SENT).astype(jnp.float32)
    msum = jnp.dot(selmask, rsum[...], preferred_element_type=jnp.float32)
    w = jax.nn.sigmoid(msum / float((TOP_K + 1) * HID))
    keep = w > 0.0
    nbr_f = jnp.concatenate(picks, axis=1)
    nbr = nbr_f.astype(jnp.int32)
    rids_i = rids.astype(jnp.int32)
    flat = nbr * N + rids_i
    idx_out[...] = jnp.concatenate([rids_i * (N + 1), flat], axis=1)
    nbr_out[...] = nbr
    vals_out[...] = jnp.where(keep, 1.0, 0.0)
    w_out[...] = jnp.where(keep, w, 0.0)


def _hbuild_body(nbrt, keepf, h_out):
    i = pl.program_id(0)
    r0 = i * BLK_C
    rids = (r0 + lax.broadcasted_iota(jnp.int32, (BLK_C, 1), 0)).astype(jnp.float32)
    cols = lax.broadcasted_iota(jnp.int32, (BLK_C, N), 1).astype(jnp.float32)
    kf = keepf[...]
    h = jnp.where(cols == rids, kf, 0.0)
    for j in range(TOP_K):
        nj = nbrt[j:j + 1, :].astype(jnp.float32)
        h = jnp.maximum(h, jnp.where(nj == rids, kf, 0.0))
    h_out[...] = h


def kernel(x0, x1, x2, mW0_1, mb0_1, mW0_2, mb0_2, mW1_1, mb1_1, mW1_2, mb1_2,
           mW2_1, mb2_1, mW2_2, mb2_2, attn_weights, fW, fb):
    f32 = jnp.float32
    aw2 = attn_weights.reshape(1, 3)
    fb2 = fb.reshape(1, HID)

    whole = lambda shape: pl.BlockSpec(shape, lambda i: (0, 0))
    rows = lambda w: pl.BlockSpec((BLK_A, w), lambda i: (i, 0))

    fused, normed, rsum = pl.pallas_call(
        _dense_body,
        grid=(N // BLK_A,),
        in_specs=[
            rows(256), rows(512), rows(128),
            whole((256, HID)), whole((HID, HID)),
            whole((512, HID)), whole((HID, HID)),
            whole((128, HID)), whole((HID, HID)),
            whole((1, 3)), whole((HID, HID)), whole((1, HID)),
        ],
        out_specs=[rows(HID), rows(HID), rows(1)],
        out_shape=[
            jax.ShapeDtypeStruct((N, HID), f32),
            jax.ShapeDtypeStruct((N, HID), f32),
            jax.ShapeDtypeStruct((N, 1), f32),
        ],
    )(x0, x1, x2, mW0_1, mW0_2, mW1_1, mW1_2, mW2_1, mW2_2, aw2, fW, fb2)

    normed_t = normed.T

    rowsB = lambda w: pl.BlockSpec((BLK_B, w), lambda i: (i, 0))
    idx, nbr, vals, w = pl.pallas_call(
        _topk_body,
        grid=(N // BLK_B,),
        in_specs=[rowsB(HID), whole((HID, N)), pl.BlockSpec((N, 1), lambda i: (0, 0))],
        out_specs=[rowsB(TOP_K + 1), rowsB(TOP_K), rowsB(1), rowsB(1)],
        out_shape=[
            jax.ShapeDtypeStruct((N, TOP_K + 1), jnp.int32),
            jax.ShapeDtypeStruct((N, TOP_K), jnp.int32),
            jax.ShapeDtypeStruct((N, 1), f32),
            jax.ShapeDtypeStruct((N, 1), f32),
        ],
    )(normed, normed_t, rsum)

    nbrt = nbr.T
    keepf = vals.reshape(1, N)

    Hmat = pl.pallas_call(
        _hbuild_body,
        grid=(N // BLK_C,),
        in_specs=[whole((TOP_K, N)), whole((1, N))],
        out_specs=pl.BlockSpec((BLK_C, N), lambda i: (i, 0)),
        out_shape=jax.ShapeDtypeStruct((N, N), f32),
    )(nbrt, keepf)

    return Hmat, w.reshape(N)
